# baseline (speedup 1.0000x reference)
import functools

import jax
import jax.numpy as jnp
from jax import lax
from jax.experimental import pallas as pl
from jax.experimental.pallas import tpu as pltpu
from jax.experimental.pallas import tpu_sc as plsc

_SC_INFO = plsc.get_sparse_core_info()
_NUM_CORES = _SC_INFO.num_cores
_NUM_SUBCORES = _SC_INFO.num_subcores
_LANES = _SC_INFO.num_lanes


def _window_sum_sc(rel_table, seq_len):
    n_rows, embed = rel_table.shape
    rows_per_w = 8
    n_active = seq_len // rows_per_w
    buf_rows = rows_per_w + seq_len

    mesh = plsc.VectorSubcoreMesh(core_axis_name="c", subcore_axis_name="s")

    @functools.partial(
        pl.kernel,
        mesh=mesh,
        out_type=jax.ShapeDtypeStruct((seq_len, embed), jnp.float32),
        scratch_types=[
            pltpu.VMEM((buf_rows, embed), jnp.float32),
            pltpu.VMEM((rows_per_w, embed), jnp.float32),
        ],
    )
    def sc_kernel(rel_hbm, out_hbm, buf_v, out_v):
        wid = lax.axis_index("s") * _NUM_CORES + lax.axis_index("c")

        @pl.when(wid < n_active)
        def _():
            base = wid * rows_per_w
            pltpu.sync_copy(rel_hbm.at[pl.ds(base, buf_rows)], buf_v)
            n_groups = embed // _LANES
            groups = [pl.ds(c * _LANES, _LANES) for c in range(n_groups)]

            def first_window(k, accs):
                return tuple(accs[c] + buf_v[k, groups[c]] for c in range(n_groups))

            accs = lax.fori_loop(
                0,
                seq_len,
                first_window,
                tuple(jnp.zeros((_LANES,), jnp.float32) for _ in range(n_groups)),
            )
            for c in range(n_groups):
                out_v[0, groups[c]] = accs[c]
            for r in range(1, rows_per_w):
                accs = tuple(
                    accs[c] - buf_v[r - 1, groups[c]] + buf_v[r + seq_len - 1, groups[c]]
                    for c in range(n_groups)
                )
                for c in range(n_groups):
                    out_v[r, groups[c]] = accs[c]
            pltpu.sync_copy(out_v, out_hbm.at[pl.ds(base, rows_per_w)])

    return sc_kernel(rel_table)


def _tc_body(cur_ref, x_ref, rel_sum_ref, temp_ref, o_ref, *, s_blk):
    i = pl.program_id(0)
    s0 = i * s_blk

    batch = x_ref.shape[1]
    cur = cur_ref[0, :][None, :, None]
    s_ids = lax.broadcasted_iota(jnp.int32, (s_blk, batch, 1), 0) + s0
    t0 = temp_ref[0, :][None, None, :]
    t1 = temp_ref[1, :][None, None, :]
    t2 = temp_ref[2, :][None, None, :]
    temporal = jnp.where(s_ids < cur, t0, jnp.where(s_ids == cur, t1, t2))

    o_ref[...] = x_ref[...] + temporal + rel_sum_ref[0][:, None, :]


def kernel(x, current_frame_idx, rel_table, temp_table):
    seq_len, batch, embed = x.shape

    s_blk = 10
    grid = seq_len // s_blk
    rel_sum = _window_sum_sc(rel_table, seq_len).reshape(grid, s_blk, embed)
    cur2 = current_frame_idx.astype(jnp.int32).reshape(1, batch)

    return pl.pallas_call(
        functools.partial(_tc_body, s_blk=s_blk),
        grid=(grid,),
        in_specs=[
            pl.BlockSpec((1, batch), lambda i: (0, 0)),
            pl.BlockSpec((s_blk, batch, embed), lambda i: (i, 0, 0)),
            pl.BlockSpec((1, s_blk, embed), lambda i: (i, 0, 0)),
            pl.BlockSpec((3, embed), lambda i: (0, 0)),
        ],
        out_specs=pl.BlockSpec((s_blk, batch, embed), lambda i: (i, 0, 0)),
        out_shape=jax.ShapeDtypeStruct((seq_len, batch, embed), x.dtype),
    )(cur2, x, rel_sum, temp_table)
